# Initial kernel scaffold; baseline (speedup 1.0000x reference)
import jax
import jax.numpy as jnp
from jax.experimental import pallas as pl


def kernel(inputs, edge_index, W1, b1, W2, b2):
    raise NotImplementedError("write your pallas kernel here")


# speedup vs baseline: 12.6489x; 12.6489x over previous
import functools

import jax
import jax.numpy as jnp
from jax import lax
from jax.experimental import pallas as pl
from jax.experimental.pallas import tpu as pltpu
from jax.experimental.pallas import tpu_sc as plsc

N = 10000
E = 320000
D_IN = 128
D_H = 128
D_OUT = 64

NC = 2
NS = 16
NW = NC * NS

E_PER_TILE = E // NW
CHUNK = 128
N_FULL = E_PER_TILE // CHUNK
TAIL = E_PER_TILE - N_FULL * CHUNK

ROWS_PER_TILE = 632
N_ACC = NS * ROWS_PER_TILE

_MESH = plsc.VectorSubcoreMesh(
    core_axis_name="c", subcore_axis_name="s", num_cores=NC, num_subcores=NS)


@functools.partial(
    pl.kernel,
    out_type=(jax.ShapeDtypeStruct((NC, N_ACC, 16), jnp.float32),
              jax.ShapeDtypeStruct((NC, N_ACC, 16), jnp.float32)),
    mesh=_MESH,
    compiler_params=pltpu.CompilerParams(use_tc_tiling_on_sc=False),
    scratch_types=[
        pltpu.VMEM((CHUNK,), jnp.int32),
        pltpu.VMEM((CHUNK,), jnp.int32),
        pltpu.VMEM((TAIL,), jnp.int32),
        pltpu.VMEM((TAIL,), jnp.int32),
        pltpu.VMEM((CHUNK, 16), jnp.float32),
        pltpu.VMEM_SHARED((N_ACC, 16), jnp.float32),
        pltpu.VMEM_SHARED((N_ACC, 16), jnp.float32),
    ],
)
def _deg_kernel(src_hbm, dst_hbm, ones_hbm, zeros_hbm, dout_hbm, din_hbm,
                sidx_v, didx_v, sidx_t, didx_t, ones_v,
                acc_out, acc_in):
  cid = lax.axis_index("c")
  sid = lax.axis_index("s")
  wid = cid * NS + sid
  base = sid * ROWS_PER_TILE

  pltpu.sync_copy(ones_hbm, ones_v)
  pltpu.sync_copy(zeros_hbm, acc_out.at[pl.ds(base, ROWS_PER_TILE)])
  pltpu.sync_copy(zeros_hbm, acc_in.at[pl.ds(base, ROWS_PER_TILE)])
  plsc.subcore_barrier()

  ebase = wid * E_PER_TILE

  def chunk_body(i, _):
    off = ebase + i * CHUNK
    pltpu.sync_copy(src_hbm.at[pl.ds(off, CHUNK)], sidx_v)
    pltpu.sync_copy(dst_hbm.at[pl.ds(off, CHUNK)], didx_v)
    pltpu.sync_copy(ones_v, acc_out.at[sidx_v], add=True)
    pltpu.sync_copy(ones_v, acc_in.at[didx_v], add=True)
    return 0

  lax.fori_loop(0, N_FULL, chunk_body, 0)

  offt = ebase + N_FULL * CHUNK
  pltpu.sync_copy(src_hbm.at[pl.ds(offt, TAIL)], sidx_t)
  pltpu.sync_copy(dst_hbm.at[pl.ds(offt, TAIL)], didx_t)
  pltpu.sync_copy(ones_v.at[pl.ds(0, TAIL)], acc_out.at[sidx_t], add=True)
  pltpu.sync_copy(ones_v.at[pl.ds(0, TAIL)], acc_in.at[didx_t], add=True)

  plsc.subcore_barrier()
  pltpu.sync_copy(acc_out.at[pl.ds(base, ROWS_PER_TILE)],
                  dout_hbm.at[cid, pl.ds(base, ROWS_PER_TILE)])
  pltpu.sync_copy(acc_in.at[pl.ds(base, ROWS_PER_TILE)],
                  din_hbm.at[cid, pl.ds(base, ROWS_PER_TILE)])


def _make_scatter_kernel(D):
  params = None if D % 128 == 0 else pltpu.CompilerParams(
      use_tc_tiling_on_sc=False)

  @functools.partial(
      pl.kernel,
      out_type=jax.ShapeDtypeStruct((NC, N_ACC, D), jnp.float32),
      mesh=_MESH,
      compiler_params=params,
      scratch_types=[
          pltpu.VMEM((CHUNK,), jnp.int32),
          pltpu.VMEM((CHUNK,), jnp.int32),
          pltpu.VMEM((TAIL,), jnp.int32),
          pltpu.VMEM((TAIL,), jnp.int32),
          pltpu.VMEM((CHUNK, D), jnp.float32),
          pltpu.VMEM((TAIL, D), jnp.float32),
          pltpu.VMEM_SHARED((N_ACC, D), jnp.float32),
          pltpu.SemaphoreType.DMA,
      ],
  )
  def _scatter_kernel(tab_hbm, src_hbm, dst_hbm, zeros_hbm, out_hbm,
                      sidx_v, didx_v, sidx_t, didx_t, rows_v, rows_t,
                      acc, sem):
    cid = lax.axis_index("c")
    sid = lax.axis_index("s")
    wid = cid * NS + sid
    base = sid * ROWS_PER_TILE

    pltpu.sync_copy(zeros_hbm, acc.at[pl.ds(base, ROWS_PER_TILE)])
    plsc.subcore_barrier()

    ebase = wid * E_PER_TILE

    def chunk_body(i, _):
      off = ebase + i * CHUNK
      pltpu.sync_copy(src_hbm.at[pl.ds(off, CHUNK)], sidx_v)
      pltpu.sync_copy(dst_hbm.at[pl.ds(off, CHUNK)], didx_v)
      pltpu.async_copy(tab_hbm.at[sidx_v], rows_v, sem).wait()
      pltpu.sync_copy(rows_v, acc.at[didx_v], add=True)
      return 0

    lax.fori_loop(0, N_FULL, chunk_body, 0)

    offt = ebase + N_FULL * CHUNK
    pltpu.sync_copy(src_hbm.at[pl.ds(offt, TAIL)], sidx_t)
    pltpu.sync_copy(dst_hbm.at[pl.ds(offt, TAIL)], didx_t)
    pltpu.async_copy(tab_hbm.at[sidx_t], rows_t, sem).wait()
    pltpu.sync_copy(rows_t, acc.at[didx_t], add=True)

    plsc.subcore_barrier()
    pltpu.sync_copy(acc.at[pl.ds(base, ROWS_PER_TILE)],
                    out_hbm.at[cid, pl.ds(base, ROWS_PER_TILE)])

  return _scatter_kernel


_scatter_128 = _make_scatter_kernel(D_H)
_scatter_64 = _make_scatter_kernel(D_OUT)


_BLK = 400
_GRID = N // _BLK


def _tc1_body(x_ref, w_ref, dout_ref, din_ref, xw_ref, rsin_ref, rsout_ref):
  rs_out = lax.rsqrt(jnp.maximum(dout_ref[0] + dout_ref[1], 1.0))
  rs_in = lax.rsqrt(jnp.maximum(din_ref[0] + din_ref[1], 1.0))
  rsout_ref[...] = rs_out
  rsin_ref[...] = rs_in
  xw = jnp.dot(x_ref[...], w_ref[...], preferred_element_type=jnp.float32)
  xw_ref[...] = xw * rs_out[:, 0:1]


def _tc1(x, w1, dout, din):
  return pl.pallas_call(
      _tc1_body,
      grid=(_GRID,),
      in_specs=[
          pl.BlockSpec((_BLK, D_IN), lambda i: (i, 0)),
          pl.BlockSpec((D_IN, D_H), lambda i: (0, 0)),
          pl.BlockSpec((NC, _BLK, 16), lambda i: (0, i, 0)),
          pl.BlockSpec((NC, _BLK, 16), lambda i: (0, i, 0)),
      ],
      out_specs=[
          pl.BlockSpec((_BLK, D_H), lambda i: (i, 0)),
          pl.BlockSpec((_BLK, 16), lambda i: (i, 0)),
          pl.BlockSpec((_BLK, 16), lambda i: (i, 0)),
      ],
      out_shape=[
          jax.ShapeDtypeStruct((N, D_H), jnp.float32),
          jax.ShapeDtypeStruct((N, 16), jnp.float32),
          jax.ShapeDtypeStruct((N, 16), jnp.float32),
      ],
  )(x, w1, dout, din)


def _tc2_body(p_ref, rsin_ref, b1_ref, w2_ref, rsout_ref, xw2_ref):
  agg = (p_ref[0] + p_ref[1]) * rsin_ref[...][:, 0:1]
  h = jnp.maximum(agg + b1_ref[...], 0.0)
  xw2 = jnp.dot(h, w2_ref[...], preferred_element_type=jnp.float32)
  xw2_ref[...] = xw2 * rsout_ref[...][:, 0:1]


def _tc2(parts1, rs_in, b1, w2, rs_out):
  return pl.pallas_call(
      _tc2_body,
      grid=(_GRID,),
      in_specs=[
          pl.BlockSpec((NC, _BLK, D_H), lambda i: (0, i, 0)),
          pl.BlockSpec((_BLK, 16), lambda i: (i, 0)),
          pl.BlockSpec((1, D_H), lambda i: (0, 0)),
          pl.BlockSpec((D_H, D_OUT), lambda i: (0, 0)),
          pl.BlockSpec((_BLK, 16), lambda i: (i, 0)),
      ],
      out_specs=pl.BlockSpec((_BLK, D_OUT), lambda i: (i, 0)),
      out_shape=jax.ShapeDtypeStruct((N, D_OUT), jnp.float32),
  )(parts1, rs_in, b1, w2, rs_out)


def _tc3_body(p_ref, rsin_ref, b2_ref, out_ref):
  agg = (p_ref[0] + p_ref[1]) * rsin_ref[...][:, 0:1]
  out_ref[...] = agg + b2_ref[...]


def _tc3(parts2, rs_in, b2):
  return pl.pallas_call(
      _tc3_body,
      grid=(_GRID,),
      in_specs=[
          pl.BlockSpec((NC, _BLK, D_OUT), lambda i: (0, i, 0)),
          pl.BlockSpec((_BLK, 16), lambda i: (i, 0)),
          pl.BlockSpec((1, D_OUT), lambda i: (0, 0)),
      ],
      out_specs=pl.BlockSpec((_BLK, D_OUT), lambda i: (i, 0)),
      out_shape=jax.ShapeDtypeStruct((N, D_OUT), jnp.float32),
  )(parts2, rs_in, b2)


def kernel(inputs, edge_index, W1, b1, W2, b2):
  src = edge_index[0]
  dst = edge_index[1]
  ones16 = jnp.ones((CHUNK, 16), jnp.float32)
  zeros16 = jnp.zeros((ROWS_PER_TILE, 16), jnp.float32)
  zeros128 = jnp.zeros((ROWS_PER_TILE, D_H), jnp.float32)
  zeros64 = jnp.zeros((ROWS_PER_TILE, D_OUT), jnp.float32)

  dout, din = _deg_kernel(src, dst, ones16, zeros16)
  xw1s, rs_in, rs_out = _tc1(inputs, W1, dout, din)
  parts1 = _scatter_128(xw1s, src, dst, zeros128)
  xw2s = _tc2(parts1, rs_in, b1.reshape(1, D_H), W2, rs_out)
  parts2 = _scatter_64(xw2s, src, dst, zeros64)
  return _tc3(parts2, rs_in, b2.reshape(1, D_OUT))


# speedup vs baseline: 16.8557x; 1.3326x over previous
import functools

import jax
import jax.numpy as jnp
from jax import lax
from jax.experimental import pallas as pl
from jax.experimental.pallas import tpu as pltpu
from jax.experimental.pallas import tpu_sc as plsc

N = 10000
E = 320000
D_IN = 128
D_H = 128
D_OUT = 64

NC = 2
NS = 16
NW = NC * NS

E_PER_TILE = E // NW
CHUNK = 128
N_FULL = E_PER_TILE // CHUNK
TAIL = E_PER_TILE - N_FULL * CHUNK

ROWS_PER_TILE = 632
N_ACC = NS * ROWS_PER_TILE

_MESH = plsc.VectorSubcoreMesh(
    core_axis_name="c", subcore_axis_name="s", num_cores=NC, num_subcores=NS)


@functools.partial(
    pl.kernel,
    out_type=(jax.ShapeDtypeStruct((NC, N_ACC, 16), jnp.float32),
              jax.ShapeDtypeStruct((NC, N_ACC, 16), jnp.float32)),
    mesh=_MESH,
    compiler_params=pltpu.CompilerParams(use_tc_tiling_on_sc=False),
    scratch_types=[
        pltpu.VMEM((CHUNK,), jnp.int32),
        pltpu.VMEM((CHUNK,), jnp.int32),
        pltpu.VMEM((TAIL,), jnp.int32),
        pltpu.VMEM((TAIL,), jnp.int32),
        pltpu.VMEM((CHUNK, 16), jnp.float32),
        pltpu.VMEM_SHARED((N_ACC, 16), jnp.float32),
        pltpu.VMEM_SHARED((N_ACC, 16), jnp.float32),
    ],
)
def _deg_kernel(src_hbm, dst_hbm, ones_hbm, zeros_hbm, dout_hbm, din_hbm,
                sidx_v, didx_v, sidx_t, didx_t, ones_v,
                acc_out, acc_in):
  cid = lax.axis_index("c")
  sid = lax.axis_index("s")
  wid = cid * NS + sid
  base = sid * ROWS_PER_TILE

  pltpu.sync_copy(ones_hbm, ones_v)
  pltpu.sync_copy(zeros_hbm, acc_out.at[pl.ds(base, ROWS_PER_TILE)])
  pltpu.sync_copy(zeros_hbm, acc_in.at[pl.ds(base, ROWS_PER_TILE)])
  plsc.subcore_barrier()

  ebase = wid * E_PER_TILE

  def chunk_body(i, _):
    off = ebase + i * CHUNK
    pltpu.sync_copy(src_hbm.at[pl.ds(off, CHUNK)], sidx_v)
    pltpu.sync_copy(dst_hbm.at[pl.ds(off, CHUNK)], didx_v)
    pltpu.sync_copy(ones_v, acc_out.at[sidx_v], add=True)
    pltpu.sync_copy(ones_v, acc_in.at[didx_v], add=True)
    return 0

  lax.fori_loop(0, N_FULL, chunk_body, 0)

  offt = ebase + N_FULL * CHUNK
  pltpu.sync_copy(src_hbm.at[pl.ds(offt, TAIL)], sidx_t)
  pltpu.sync_copy(dst_hbm.at[pl.ds(offt, TAIL)], didx_t)
  pltpu.sync_copy(ones_v.at[pl.ds(0, TAIL)], acc_out.at[sidx_t], add=True)
  pltpu.sync_copy(ones_v.at[pl.ds(0, TAIL)], acc_in.at[didx_t], add=True)

  plsc.subcore_barrier()
  pltpu.sync_copy(acc_out.at[pl.ds(base, ROWS_PER_TILE)],
                  dout_hbm.at[cid, pl.ds(base, ROWS_PER_TILE)])
  pltpu.sync_copy(acc_in.at[pl.ds(base, ROWS_PER_TILE)],
                  din_hbm.at[cid, pl.ds(base, ROWS_PER_TILE)])


def _make_scatter_kernel(D):
  params = None if D % 128 == 0 else pltpu.CompilerParams(
      use_tc_tiling_on_sc=False)

  @functools.partial(
      pl.kernel,
      out_type=jax.ShapeDtypeStruct((NC, N_ACC, D), jnp.float32),
      mesh=_MESH,
      compiler_params=params,
      scratch_types=[
          pltpu.VMEM((CHUNK,), jnp.int32),
          pltpu.VMEM((CHUNK,), jnp.int32),
          pltpu.VMEM((CHUNK,), jnp.int32),
          pltpu.VMEM((CHUNK,), jnp.int32),
          pltpu.VMEM((TAIL,), jnp.int32),
          pltpu.VMEM((TAIL,), jnp.int32),
          pltpu.VMEM((CHUNK, D), jnp.float32),
          pltpu.VMEM((CHUNK, D), jnp.float32),
          pltpu.VMEM((TAIL, D), jnp.float32),
          pltpu.VMEM_SHARED((N_ACC, D), jnp.float32),
          pltpu.SemaphoreType.DMA,
          pltpu.SemaphoreType.DMA,
      ],
  )
  def _scatter_kernel(tab_hbm, src_hbm, dst_hbm, zeros_hbm, out_hbm,
                      sidx0, sidx1, didx0, didx1, sidx_t, didx_t,
                      rows0, rows1, rows_t, acc, sem0, sem1):
    cid = lax.axis_index("c")
    sid = lax.axis_index("s")
    wid = cid * NS + sid
    base = sid * ROWS_PER_TILE
    ebase = wid * E_PER_TILE

    sidx = (sidx0, sidx1)
    didx = (didx0, didx1)
    rows = (rows0, rows1)
    sems = (sem0, sem1)

    def start_gather(i, slot):
      off = ebase + i * CHUNK
      pltpu.sync_copy(src_hbm.at[pl.ds(off, CHUNK)], sidx[slot])
      pltpu.sync_copy(dst_hbm.at[pl.ds(off, CHUNK)], didx[slot])
      return pltpu.async_copy(tab_hbm.at[sidx[slot]], rows[slot], sems[slot])

    def drain(slot):
      pltpu.make_async_copy(tab_hbm.at[sidx[slot]], rows[slot],
                            sems[slot]).wait()
      pltpu.sync_copy(rows[slot], acc.at[didx[slot]], add=True)

    start_gather(0, 0)
    pltpu.sync_copy(zeros_hbm, acc.at[pl.ds(base, ROWS_PER_TILE)])
    plsc.subcore_barrier()

    def loop_body(g, _):
      start_gather(2 * g + 1, 1)
      drain(0)

      @pl.when(g < N_FULL // 2 - 1)
      def _():
        start_gather(2 * g + 2, 0)

      drain(1)
      return 0

    lax.fori_loop(0, N_FULL // 2, loop_body, 0)

    offt = ebase + N_FULL * CHUNK
    pltpu.sync_copy(src_hbm.at[pl.ds(offt, TAIL)], sidx_t)
    pltpu.sync_copy(dst_hbm.at[pl.ds(offt, TAIL)], didx_t)
    pltpu.async_copy(tab_hbm.at[sidx_t], rows_t, sem0).wait()
    pltpu.sync_copy(rows_t, acc.at[didx_t], add=True)

    plsc.subcore_barrier()
    pltpu.sync_copy(acc.at[pl.ds(base, ROWS_PER_TILE)],
                    out_hbm.at[cid, pl.ds(base, ROWS_PER_TILE)])

  return _scatter_kernel


_scatter_128 = _make_scatter_kernel(D_H)
_scatter_64 = _make_scatter_kernel(D_OUT)


_BLK = 400
_GRID = N // _BLK


def _tc1_body(x_ref, w_ref, dout_ref, din_ref, xw_ref, rsin_ref, rsout_ref):
  rs_out = lax.rsqrt(jnp.maximum(dout_ref[0] + dout_ref[1], 1.0))
  rs_in = lax.rsqrt(jnp.maximum(din_ref[0] + din_ref[1], 1.0))
  rsout_ref[...] = rs_out
  rsin_ref[...] = rs_in
  xw = jnp.dot(x_ref[...], w_ref[...], preferred_element_type=jnp.float32)
  xw_ref[...] = xw * rs_out[:, 0:1]


def _tc1(x, w1, dout, din):
  return pl.pallas_call(
      _tc1_body,
      grid=(_GRID,),
      in_specs=[
          pl.BlockSpec((_BLK, D_IN), lambda i: (i, 0)),
          pl.BlockSpec((D_IN, D_H), lambda i: (0, 0)),
          pl.BlockSpec((NC, _BLK, 16), lambda i: (0, i, 0)),
          pl.BlockSpec((NC, _BLK, 16), lambda i: (0, i, 0)),
      ],
      out_specs=[
          pl.BlockSpec((_BLK, D_H), lambda i: (i, 0)),
          pl.BlockSpec((_BLK, 16), lambda i: (i, 0)),
          pl.BlockSpec((_BLK, 16), lambda i: (i, 0)),
      ],
      out_shape=[
          jax.ShapeDtypeStruct((N, D_H), jnp.float32),
          jax.ShapeDtypeStruct((N, 16), jnp.float32),
          jax.ShapeDtypeStruct((N, 16), jnp.float32),
      ],
  )(x, w1, dout, din)


def _tc2_body(p_ref, rsin_ref, b1_ref, w2_ref, rsout_ref, xw2_ref):
  agg = (p_ref[0] + p_ref[1]) * rsin_ref[...][:, 0:1]
  h = jnp.maximum(agg + b1_ref[...], 0.0)
  xw2 = jnp.dot(h, w2_ref[...], preferred_element_type=jnp.float32)
  xw2_ref[...] = xw2 * rsout_ref[...][:, 0:1]


def _tc2(parts1, rs_in, b1, w2, rs_out):
  return pl.pallas_call(
      _tc2_body,
      grid=(_GRID,),
      in_specs=[
          pl.BlockSpec((NC, _BLK, D_H), lambda i: (0, i, 0)),
          pl.BlockSpec((_BLK, 16), lambda i: (i, 0)),
          pl.BlockSpec((1, D_H), lambda i: (0, 0)),
          pl.BlockSpec((D_H, D_OUT), lambda i: (0, 0)),
          pl.BlockSpec((_BLK, 16), lambda i: (i, 0)),
      ],
      out_specs=pl.BlockSpec((_BLK, D_OUT), lambda i: (i, 0)),
      out_shape=jax.ShapeDtypeStruct((N, D_OUT), jnp.float32),
  )(parts1, rs_in, b1, w2, rs_out)


def _tc3_body(p_ref, rsin_ref, b2_ref, out_ref):
  agg = (p_ref[0] + p_ref[1]) * rsin_ref[...][:, 0:1]
  out_ref[...] = agg + b2_ref[...]


def _tc3(parts2, rs_in, b2):
  return pl.pallas_call(
      _tc3_body,
      grid=(_GRID,),
      in_specs=[
          pl.BlockSpec((NC, _BLK, D_OUT), lambda i: (0, i, 0)),
          pl.BlockSpec((_BLK, 16), lambda i: (i, 0)),
          pl.BlockSpec((1, D_OUT), lambda i: (0, 0)),
      ],
      out_specs=pl.BlockSpec((_BLK, D_OUT), lambda i: (i, 0)),
      out_shape=jax.ShapeDtypeStruct((N, D_OUT), jnp.float32),
  )(parts2, rs_in, b2)


def kernel(inputs, edge_index, W1, b1, W2, b2):
  src = edge_index[0]
  dst = edge_index[1]
  ones16 = jnp.ones((CHUNK, 16), jnp.float32)
  zeros16 = jnp.zeros((ROWS_PER_TILE, 16), jnp.float32)
  zeros128 = jnp.zeros((ROWS_PER_TILE, D_H), jnp.float32)
  zeros64 = jnp.zeros((ROWS_PER_TILE, D_OUT), jnp.float32)

  dout, din = _deg_kernel(src, dst, ones16, zeros16)
  xw1s, rs_in, rs_out = _tc1(inputs, W1, dout, din)
  parts1 = _scatter_128(xw1s, src, dst, zeros128)
  xw2s = _tc2(parts1, rs_in, b1.reshape(1, D_H), W2, rs_out)
  parts2 = _scatter_64(xw2s, src, dst, zeros64)
  return _tc3(parts2, rs_in, b2.reshape(1, D_OUT))
